# baseline (device time: 438385 ns/iter reference)
import jax
import jax.numpy as jnp
from jax import lax
from jax.experimental import pallas as pl
from jax.experimental.pallas import tpu as pltpu

N_DEV = 16

_HAM = [0, 1, 5, 4, 8, 9, 13, 12, 15, 14, 10, 11, 7, 6, 2, 3]
_K = [0] * N_DEV
for _i, _p in enumerate(_HAM):
    _K[_p] = _i
_SUCC = [_HAM[(_K[p] + 1) % N_DEV] for p in range(N_DEV)]
_PRED = [_HAM[(_K[p] - 1) % N_DEV] for p in range(N_DEV)]


def _lut(idx, table):
    import jax.numpy as jnp
    acc = jnp.int32(table[0])
    for p in range(1, N_DEV):
        acc = jnp.where(idx == p, jnp.int32(table[p]), acc)
    return acc


def kernel(x, W):
    t, d = x.shape
    _, v_per = W.shape
    v_total = N_DEV * v_per

    def body(x_ref, w_ref, dummy_ref, out_ref, cbuf, msbuf,
             ms_send, ms_recv, ring_send, ring_recv,
             ring2_send, ring2_recv, csem):
        my = lax.axis_index("i")
        right = _lut(my, _SUCC)
        left = _lut(my, _PRED)

        barrier_sem = pltpu.get_barrier_semaphore()
        for k in range(1, N_DEV):
            pl.semaphore_signal(
                barrier_sem, inc=1,
                device_id=(lax.rem(my + k, N_DEV),),
                device_id_type=pl.DeviceIdType.MESH,
            )

        logits = jnp.dot(
            x_ref[:, :], w_ref[:, :], preferred_element_type=jnp.float32
        )
        m_loc = jnp.max(logits, axis=-1, keepdims=True)
        e_loc = jnp.exp(logits - m_loc)
        s_loc = jnp.sum(e_loc, axis=-1, keepdims=True)
        cbuf[0] = e_loc

        pad = jnp.zeros((t, 126), dtype=jnp.float32)
        msbuf[my] = jnp.concatenate([m_loc, s_loc, pad], axis=-1)

        pl.semaphore_wait(barrier_sem, N_DEV - 1)

        sends = []
        for k in range(1, N_DEV):
            tgt = lax.rem(my + k, N_DEV)
            r = pltpu.make_async_remote_copy(
                src_ref=msbuf.at[my],
                dst_ref=msbuf.at[my],
                send_sem=ms_send.at[k - 1],
                recv_sem=ms_recv.at[my],
                device_id=(tgt,),
                device_id_type=pl.DeviceIdType.MESH,
            )
            r.start()
            sends.append(r)
        for k in range(1, N_DEV):
            src_dev = lax.rem(my - k + N_DEV, N_DEV)
            recv = pltpu.make_async_remote_copy(
                src_ref=msbuf.at[src_dev],
                dst_ref=msbuf.at[src_dev],
                send_sem=ms_send.at[k - 1],
                recv_sem=ms_recv.at[src_dev],
                device_id=(my,),
                device_id_type=pl.DeviceIdType.MESH,
            )
            recv.wait_recv()
        for r in sends:
            r.wait_send()

        M = msbuf[0, :, 0:1]
        for c in range(1, N_DEV):
            M = jnp.maximum(M, msbuf[c, :, 0:1])
        S = jnp.zeros((t, 1), dtype=jnp.float32)
        for c in range(N_DEV):
            S = S + msbuf[c, :, 1:2] * jnp.exp(msbuf[c, :, 0:1] - M)
        cbuf[0] = cbuf[0] * (jnp.exp(m_loc - M) * (1.0 / S))
        cp = pltpu.make_async_copy(
            cbuf.at[0], out_ref.at[:, pl.ds(my * v_per, v_per)], csem
        )
        cp.start()

        v_half = v_per // 2
        n_sub = 4
        v_q = v_half // n_sub

        def mk_hop(h, direction, q):
            if direction == 0:
                origin = _lut(
                    my, [_HAM[(_K[p] - h) % N_DEV] for p in range(N_DEV)]
                )
                col = q * v_q
                tgt = right
                ssem, rsem = ring_send, ring_recv
            else:
                origin = _lut(
                    my, [_HAM[(_K[p] + h) % N_DEV] for p in range(N_DEV)]
                )
                col = v_half + q * v_q
                tgt = left
                ssem, rsem = ring2_send, ring2_recv
            sl = pl.ds(origin * v_per + col, v_q)
            if h == 0:
                src = cbuf.at[0, :, pl.ds(col, v_q)]
            else:
                src = out_ref.at[:, sl]
            return pltpu.make_async_remote_copy(
                src_ref=src,
                dst_ref=out_ref.at[:, sl],
                send_sem=ssem.at[h, q],
                recv_sem=rsem.at[h, q],
                device_id=(tgt,),
                device_id_type=pl.DeviceIdType.MESH,
            )

        streams = {}
        for q in range(n_sub):
            for direction in range(2):
                r = mk_hop(0, direction, q)
                r.start()
                streams[(direction, q)] = [r]
        for h in range(N_DEV - 1):
            for q in range(n_sub):
                for direction in range(2):
                    streams[(direction, q)][h].wait_recv()
                    if h < N_DEV - 2:
                        r = mk_hop(h + 1, direction, q)
                        r.start()
                        streams[(direction, q)].append(r)
        for lst in streams.values():
            for r in lst:
                r.wait_send()
        cp.wait()

    dummy = jnp.zeros((t, v_total), jnp.float32)
    return pl.pallas_call(
        body,
        out_shape=jax.ShapeDtypeStruct((t, v_total), jnp.float32),
        in_specs=[
            pl.BlockSpec(memory_space=pltpu.VMEM),
            pl.BlockSpec(memory_space=pltpu.VMEM),
            pl.BlockSpec(memory_space=pl.ANY),
        ],
        out_specs=pl.BlockSpec(memory_space=pl.ANY),
        input_output_aliases={2: 0},
        scratch_shapes=[
            pltpu.VMEM((1, t, v_per), jnp.float32),
            pltpu.VMEM((N_DEV, t, 128), jnp.float32),
            pltpu.SemaphoreType.DMA((N_DEV - 1,)),
            pltpu.SemaphoreType.DMA((N_DEV,)),
            pltpu.SemaphoreType.DMA((N_DEV - 1, 4)),
            pltpu.SemaphoreType.DMA((N_DEV - 1, 4)),
            pltpu.SemaphoreType.DMA((N_DEV - 1, 4)),
            pltpu.SemaphoreType.DMA((N_DEV - 1, 4)),
            pltpu.SemaphoreType.DMA,
        ],
        compiler_params=pltpu.CompilerParams(collective_id=0),
    )(x, W, dummy)


# device time: 419571 ns/iter; 1.0448x vs baseline; 1.0448x over previous
import jax
import jax.numpy as jnp
from jax import lax
from jax.experimental import pallas as pl
from jax.experimental.pallas import tpu as pltpu

N_DEV = 16

_HAM = [0, 1, 5, 4, 8, 9, 13, 12, 15, 14, 10, 11, 7, 6, 2, 3]
_K = [0] * N_DEV
for _i, _p in enumerate(_HAM):
    _K[_p] = _i
_SUCC = [_HAM[(_K[p] + 1) % N_DEV] for p in range(N_DEV)]
_PRED = [_HAM[(_K[p] - 1) % N_DEV] for p in range(N_DEV)]


def _lut(idx, table):
    import jax.numpy as jnp
    acc = jnp.int32(table[0])
    for p in range(1, N_DEV):
        acc = jnp.where(idx == p, jnp.int32(table[p]), acc)
    return acc


def kernel(x, W):
    t, d = x.shape
    _, v_per = W.shape
    v_total = N_DEV * v_per

    def body(x_ref, w_ref, out_ref, cbuf, msbuf,
             ms_send, ms_recv, ring_send, ring_recv,
             ring2_send, ring2_recv, csem):
        my = lax.axis_index("i")
        right = _lut(my, _SUCC)
        left = _lut(my, _PRED)

        barrier_sem = pltpu.get_barrier_semaphore()
        for k in range(1, N_DEV):
            pl.semaphore_signal(
                barrier_sem, inc=1,
                device_id=(lax.rem(my + k, N_DEV),),
                device_id_type=pl.DeviceIdType.MESH,
            )

        logits = jnp.dot(
            x_ref[:, :], w_ref[:, :], preferred_element_type=jnp.float32
        )
        m_loc = jnp.max(logits, axis=-1, keepdims=True)
        e_loc = jnp.exp(logits - m_loc)
        s_loc = jnp.sum(e_loc, axis=-1, keepdims=True)
        cbuf[0] = e_loc

        pad = jnp.zeros((t, 126), dtype=jnp.float32)
        msbuf[my] = jnp.concatenate([m_loc, s_loc, pad], axis=-1)

        pl.semaphore_wait(barrier_sem, N_DEV - 1)

        sends = []
        for k in range(1, N_DEV):
            tgt = lax.rem(my + k, N_DEV)
            r = pltpu.make_async_remote_copy(
                src_ref=msbuf.at[my],
                dst_ref=msbuf.at[my],
                send_sem=ms_send.at[k - 1],
                recv_sem=ms_recv.at[my],
                device_id=(tgt,),
                device_id_type=pl.DeviceIdType.MESH,
            )
            r.start()
            sends.append(r)
        for k in range(1, N_DEV):
            src_dev = lax.rem(my - k + N_DEV, N_DEV)
            recv = pltpu.make_async_remote_copy(
                src_ref=msbuf.at[src_dev],
                dst_ref=msbuf.at[src_dev],
                send_sem=ms_send.at[k - 1],
                recv_sem=ms_recv.at[src_dev],
                device_id=(my,),
                device_id_type=pl.DeviceIdType.MESH,
            )
            recv.wait_recv()
        for r in sends:
            r.wait_send()

        M = msbuf[0, :, 0:1]
        for c in range(1, N_DEV):
            M = jnp.maximum(M, msbuf[c, :, 0:1])
        S = jnp.zeros((t, 1), dtype=jnp.float32)
        for c in range(N_DEV):
            S = S + msbuf[c, :, 1:2] * jnp.exp(msbuf[c, :, 0:1] - M)
        cbuf[0] = cbuf[0] * (jnp.exp(m_loc - M) * (1.0 / S))
        cp = pltpu.make_async_copy(
            cbuf.at[0], out_ref.at[:, pl.ds(my * v_per, v_per)], csem
        )
        cp.start()

        v_half = v_per // 2
        n_sub = 4
        v_q = v_half // n_sub

        def mk_hop(h, direction, q):
            if direction == 0:
                origin = _lut(
                    my, [_HAM[(_K[p] - h) % N_DEV] for p in range(N_DEV)]
                )
                col = q * v_q
                tgt = right
                ssem, rsem = ring_send, ring_recv
            else:
                origin = _lut(
                    my, [_HAM[(_K[p] + h) % N_DEV] for p in range(N_DEV)]
                )
                col = v_half + q * v_q
                tgt = left
                ssem, rsem = ring2_send, ring2_recv
            sl = pl.ds(origin * v_per + col, v_q)
            if h == 0:
                src = cbuf.at[0, :, pl.ds(col, v_q)]
            else:
                src = out_ref.at[:, sl]
            return pltpu.make_async_remote_copy(
                src_ref=src,
                dst_ref=out_ref.at[:, sl],
                send_sem=ssem.at[h, q],
                recv_sem=rsem.at[h, q],
                device_id=(tgt,),
                device_id_type=pl.DeviceIdType.MESH,
            )

        streams = {}
        for q in range(n_sub):
            for direction in range(2):
                r = mk_hop(0, direction, q)
                r.start()
                streams[(direction, q)] = [r]
        for h in range(N_DEV - 1):
            for q in range(n_sub):
                for direction in range(2):
                    streams[(direction, q)][h].wait_recv()
                    if h < N_DEV - 2:
                        r = mk_hop(h + 1, direction, q)
                        r.start()
                        streams[(direction, q)].append(r)
        for lst in streams.values():
            for r in lst:
                r.wait_send()
        cp.wait()

    return pl.pallas_call(
        body,
        out_shape=jax.ShapeDtypeStruct((t, v_total), jnp.float32),
        in_specs=[
            pl.BlockSpec(memory_space=pltpu.VMEM),
            pl.BlockSpec(memory_space=pltpu.VMEM),
        ],
        out_specs=pl.BlockSpec(memory_space=pl.ANY),
        scratch_shapes=[
            pltpu.VMEM((1, t, v_per), jnp.float32),
            pltpu.VMEM((N_DEV, t, 128), jnp.float32),
            pltpu.SemaphoreType.DMA((N_DEV - 1,)),
            pltpu.SemaphoreType.DMA((N_DEV,)),
            pltpu.SemaphoreType.DMA((N_DEV - 1, 4)),
            pltpu.SemaphoreType.DMA((N_DEV - 1, 4)),
            pltpu.SemaphoreType.DMA((N_DEV - 1, 4)),
            pltpu.SemaphoreType.DMA((N_DEV - 1, 4)),
            pltpu.SemaphoreType.DMA,
        ],
        compiler_params=pltpu.CompilerParams(collective_id=0),
    )(x, W)
